# baseline (device time: 34810 ns/iter reference)
import jax
import jax.numpy as jnp
from jax import lax
from jax.experimental import pallas as pl
from jax.experimental.pallas import tpu as pltpu

N_DEV = 32
B, Sq, Hq, Hkv, Dh = 2, 256, 8, 2, 64
D_MODEL = 768
DQ = Hq * Dh
GROUP = Hq // Hkv
DE = Dh + 8
DQE = Hq * DE
R1 = 8
R2 = 4
BLK = Sq // R1
SEG = BLK // R2
SCALE = 0.125

BF16 = jnp.bfloat16
F32 = jnp.float32


def kernel(x, Wq, Wo, K_ext, V_ext):
    skv = K_ext.shape[1]

    def body(
        x_ref, wq_ref, wo_ref, k_ref, v_ref, out_ref,
        q_buf, acc, send, rs1_recv, rs2_recv, norm_buf,
        rs1_ssem, rs1_rsem, rs2_ssem, rs2_rsem,
        ag1_ssem, ag1_rsem, ag2_ssem, ag2_rsem,
    ):
        me = lax.axis_index("i")
        partners = [me ^ j for j in range(1, R1)] + [
            me ^ (R1 * j) for j in range(1, R2)
        ]

        barrier = pltpu.get_barrier_semaphore()
        for p in partners:
            pl.semaphore_signal(
                barrier, inc=1,
                device_id=(p,), device_id_type=pl.DeviceIdType.MESH,
            )

        wq = (wq_ref[...] * (SCALE * 1.4426950408889634)).astype(BF16)
        for b in range(B):
            xb = x_ref[b].astype(BF16)
            for h in range(Hq):
                q_buf[b, h * Sq:(h + 1) * Sq, :] = lax.dot_general(
                    xb, wq[:, h * Dh:(h + 1) * Dh],
                    (((1,), (0,)), ((), ())),
                    preferred_element_type=F32,
                ).astype(BF16)

        lane = lax.broadcasted_iota(jnp.int32, (skv, DE - Dh), 1)
        ext = (lane == 0).astype(BF16)

        def compute_partials(b):
            for g in range(Hkv):
                kk = k_ref[b, :, g, :].astype(BF16)
                vve = jnp.concatenate(
                    [v_ref[b, :, g, :].astype(BF16), ext], axis=1
                )
                qs = q_buf[b, g * GROUP * Sq:(g + 1) * GROUP * Sq, :]
                s = lax.dot_general(
                    qs, kk, (((1,), (1,)), ((), ())),
                    preferred_element_type=F32,
                )
                p = jnp.exp2(s).astype(BF16)
                oe = lax.dot_general(
                    p, vve, (((1,), (0,)), ((), ())),
                    preferred_element_type=F32,
                )
                for i, h in enumerate(range(g * GROUP, (g + 1) * GROUP)):
                    acc[b, :, h * DE:(h + 1) * DE] = (
                        oe[i * Sq:(i + 1) * Sq, :]
                    )

        blk = (me & (R1 - 1)) * BLK
        sub = ((me >> 3) & (R2 - 1)) * SEG
        seg = blk + sub

        def exchange(src_ref, dst_ref, ssem, rsem, p):
            d = pltpu.make_async_remote_copy(
                src_ref=src_ref, dst_ref=dst_ref,
                send_sem=ssem, recv_sem=rsem,
                device_id=(p,), device_id_type=pl.DeviceIdType.MESH,
            )
            d.start()
            return d

        def start_rs1(c):
            send[c] = acc[c].astype(BF16)
            ds = []
            for j in range(1, R1):
                p = me ^ j
                off = (p & (R1 - 1)) * BLK
                ds.append(exchange(
                    send.at[c, pl.ds(off, BLK), :],
                    rs1_recv.at[c, j], rs1_ssem.at[c, j], rs1_rsem.at[c, j],
                    p,
                ))
            return ds

        def finish_rs1(c, ds):
            for d in ds:
                d.wait()
            v = acc[c, pl.ds(blk, BLK), :]
            for j in range(1, R1):
                v = v + rs1_recv[c, j].astype(F32)
            acc[c, pl.ds(blk, BLK), :] = v

        def start_rs2(c):
            send[c, pl.ds(0, BLK), :] = acc[c, pl.ds(blk, BLK), :].astype(BF16)
            ds = []
            for j in range(1, R2):
                p = me ^ (R1 * j)
                off = (((me >> 3) ^ j) & (R2 - 1)) * SEG
                ds.append(exchange(
                    send.at[c, pl.ds(off, SEG), :],
                    rs2_recv.at[c, j], rs2_ssem.at[c, j], rs2_rsem.at[c, j],
                    p,
                ))
            return ds

        def finish_rs2(c, ds):
            for d in ds:
                d.wait()
            v = acc[c, pl.ds(seg, SEG), :]
            for j in range(1, R2):
                v = v + rs2_recv[c, j].astype(F32)
            acc[c, pl.ds(seg, SEG), :] = v
            for h in range(Hq):
                o = acc[c, pl.ds(seg, SEG), h * DE:h * DE + Dh]
                l = acc[c, pl.ds(seg, SEG), h * DE + Dh:h * DE + Dh + 1]
                norm_buf[c, pl.ds(seg, SEG), h * Dh:(h + 1) * Dh] = (
                    (o / l).astype(BF16)
                )

        def start_ag1(c):
            return [
                exchange(
                    norm_buf.at[c, pl.ds(seg, SEG), :],
                    norm_buf.at[c, pl.ds(seg, SEG), :],
                    ag1_ssem.at[c, j], ag1_rsem.at[c, j],
                    me ^ (R1 * j),
                )
                for j in range(1, R2)
            ]

        def start_ag2(c):
            return [
                exchange(
                    norm_buf.at[c, pl.ds(blk, BLK), :],
                    norm_buf.at[c, pl.ds(blk, BLK), :],
                    ag2_ssem.at[c, j], ag2_rsem.at[c, j],
                    me ^ j,
                )
                for j in range(1, R1)
            ]

        wo = wo_ref[...].astype(BF16)

        def project(b):
            out_ref[b] = lax.dot_general(
                norm_buf[b], wo, (((1,), (0,)), ((), ())),
                preferred_element_type=F32,
            )

        compute_partials(0)
        pl.semaphore_wait(barrier, len(partners))
        dA = start_rs1(0)
        compute_partials(1)
        dB = start_rs1(1)
        finish_rs1(0, dA)
        dA = start_rs2(0)
        finish_rs1(1, dB)
        dB = start_rs2(1)
        finish_rs2(0, dA)
        dA = start_ag1(0)
        finish_rs2(1, dB)
        dB = start_ag1(1)
        for d in dA:
            d.wait()
        dA = start_ag2(0)
        for d in dB:
            d.wait()
        dB = start_ag2(1)
        for d in dA:
            d.wait()
        project(0)
        for d in dB:
            d.wait()
        project(1)

    return pl.pallas_call(
        body,
        out_shape=jax.ShapeDtypeStruct((B, Sq, D_MODEL), F32),
        in_specs=[pl.BlockSpec(memory_space=pltpu.VMEM)] * 5,
        out_specs=pl.BlockSpec(memory_space=pltpu.VMEM),
        scratch_shapes=[
            pltpu.VMEM((B, Hq * Sq, Dh), BF16),
            pltpu.VMEM((B, Sq, DQE), F32),
            pltpu.VMEM((B, Sq, DQE), BF16),
            pltpu.VMEM((B, R1, BLK, DQE), BF16),
            pltpu.VMEM((B, R2, SEG, DQE), BF16),
            pltpu.VMEM((B, Sq, DQ), BF16),
            pltpu.SemaphoreType.DMA((B, R1)),
            pltpu.SemaphoreType.DMA((B, R1)),
            pltpu.SemaphoreType.DMA((B, R2)),
            pltpu.SemaphoreType.DMA((B, R2)),
            pltpu.SemaphoreType.DMA((B, R2)),
            pltpu.SemaphoreType.DMA((B, R2)),
            pltpu.SemaphoreType.DMA((B, R1)),
            pltpu.SemaphoreType.DMA((B, R1)),
        ],
        compiler_params=pltpu.CompilerParams(collective_id=0),
    )(x, Wq, Wo, K_ext, V_ext)


# device time: 33995 ns/iter; 1.0240x vs baseline; 1.0240x over previous
import jax
import jax.numpy as jnp
from jax import lax
from jax.experimental import pallas as pl
from jax.experimental.pallas import tpu as pltpu

N_DEV = 32
B, Sq, Hq, Hkv, Dh = 2, 256, 8, 2, 64
D_MODEL = 768
DQ = Hq * Dh
GROUP = Hq // Hkv
DE = Dh + 8
DQE = Hq * DE
R1 = 8
R2 = 4
BLK = Sq // R1
SEG = BLK // R2
SCALE = 0.125

BF16 = jnp.bfloat16
F32 = jnp.float32


def kernel(x, Wq, Wo, K_ext, V_ext):
    skv = K_ext.shape[1]

    def body(
        x_ref, wq_ref, wo_ref, k_ref, v_ref, out_ref,
        q_buf, acc, send, rs1_recv, rs2_recv, norm_buf,
        rs1_ssem, rs1_rsem, rs2_ssem, rs2_rsem,
        ag1_ssem, ag1_rsem, ag2_ssem, ag2_rsem,
    ):
        me = lax.axis_index("i")
        partners = [me ^ j for j in range(1, R1)] + [
            me ^ (R1 * j) for j in range(1, R2)
        ]

        barrier = pltpu.get_barrier_semaphore()
        for p in partners:
            pl.semaphore_signal(
                barrier, inc=1,
                device_id=(p,), device_id_type=pl.DeviceIdType.MESH,
            )

        wq = (wq_ref[...] * (SCALE * 1.4426950408889634)).astype(BF16)
        q_buf[...] = lax.dot_general(
            x_ref[...].reshape(B * Sq, D_MODEL).astype(BF16), wq,
            (((1,), (0,)), ((), ())),
            preferred_element_type=F32,
        ).astype(BF16).reshape(B, Sq, DQ)

        lane = lax.broadcasted_iota(jnp.int32, (skv, DE - Dh), 1)
        ext = (lane == 0).astype(BF16)

        def compute_partials(b):
            for g in range(Hkv):
                kk = k_ref[b, :, g, :].astype(BF16)
                vve = jnp.concatenate(
                    [v_ref[b, :, g, :].astype(BF16), ext], axis=1
                )
                for h in range(g * GROUP, (g + 1) * GROUP):
                    q = q_buf[b, :, h * Dh:(h + 1) * Dh]
                    s = lax.dot_general(
                        q, kk, (((1,), (1,)), ((), ())),
                        preferred_element_type=F32,
                    )
                    p = jnp.exp2(s)
                    acc[b, :, h * DE:(h + 1) * DE] = lax.dot_general(
                        p.astype(BF16), vve, (((1,), (0,)), ((), ())),
                        preferred_element_type=F32,
                    )

        blk = (me & (R1 - 1)) * BLK
        sub = ((me >> 3) & (R2 - 1)) * SEG
        seg = blk + sub

        def exchange(src_ref, dst_ref, ssem, rsem, p):
            d = pltpu.make_async_remote_copy(
                src_ref=src_ref, dst_ref=dst_ref,
                send_sem=ssem, recv_sem=rsem,
                device_id=(p,), device_id_type=pl.DeviceIdType.MESH,
            )
            d.start()
            return d

        def start_rs1(c):
            send[c] = acc[c].astype(BF16)
            ds = []
            for j in range(1, R1):
                p = me ^ j
                off = (p & (R1 - 1)) * BLK
                ds.append(exchange(
                    send.at[c, pl.ds(off, BLK), :],
                    rs1_recv.at[c, j], rs1_ssem.at[c, j], rs1_rsem.at[c, j],
                    p,
                ))
            return ds

        def finish_rs1(c, ds):
            for d in ds:
                d.wait()
            v = acc[c, pl.ds(blk, BLK), :]
            for j in range(1, R1):
                v = v + rs1_recv[c, j].astype(F32)
            acc[c, pl.ds(blk, BLK), :] = v

        def start_rs2(c):
            send[c, pl.ds(0, BLK), :] = acc[c, pl.ds(blk, BLK), :].astype(BF16)
            ds = []
            for j in range(1, R2):
                p = me ^ (R1 * j)
                off = (((me >> 3) ^ j) & (R2 - 1)) * SEG
                ds.append(exchange(
                    send.at[c, pl.ds(off, SEG), :],
                    rs2_recv.at[c, j], rs2_ssem.at[c, j], rs2_rsem.at[c, j],
                    p,
                ))
            return ds

        def finish_rs2(c, ds):
            for d in ds:
                d.wait()
            v = acc[c, pl.ds(seg, SEG), :]
            for j in range(1, R2):
                v = v + rs2_recv[c, j].astype(F32)
            acc[c, pl.ds(seg, SEG), :] = v
            for h in range(Hq):
                o = acc[c, pl.ds(seg, SEG), h * DE:h * DE + Dh]
                l = acc[c, pl.ds(seg, SEG), h * DE + Dh:h * DE + Dh + 1]
                norm_buf[c, pl.ds(seg, SEG), h * Dh:(h + 1) * Dh] = (
                    (o / l).astype(BF16)
                )

        def start_ag1(c):
            return [
                exchange(
                    norm_buf.at[c, pl.ds(seg, SEG), :],
                    norm_buf.at[c, pl.ds(seg, SEG), :],
                    ag1_ssem.at[c, j], ag1_rsem.at[c, j],
                    me ^ (R1 * j),
                )
                for j in range(1, R2)
            ]

        def start_ag2(c):
            return [
                exchange(
                    norm_buf.at[c, pl.ds(blk, BLK), :],
                    norm_buf.at[c, pl.ds(blk, BLK), :],
                    ag2_ssem.at[c, j], ag2_rsem.at[c, j],
                    me ^ j,
                )
                for j in range(1, R1)
            ]

        wo = wo_ref[...].astype(BF16)

        def project(b):
            out_ref[b] = lax.dot_general(
                norm_buf[b], wo, (((1,), (0,)), ((), ())),
                preferred_element_type=F32,
            )

        compute_partials(0)
        pl.semaphore_wait(barrier, len(partners))
        dA = start_rs1(0)
        compute_partials(1)
        dB = start_rs1(1)
        finish_rs1(0, dA)
        dA = start_rs2(0)
        finish_rs1(1, dB)
        dB = start_rs2(1)
        finish_rs2(0, dA)
        dA = start_ag1(0)
        finish_rs2(1, dB)
        dB = start_ag1(1)
        for d in dA:
            d.wait()
        dA = start_ag2(0)
        for d in dB:
            d.wait()
        dB = start_ag2(1)
        for d in dA:
            d.wait()
        project(0)
        for d in dB:
            d.wait()
        project(1)

    return pl.pallas_call(
        body,
        out_shape=jax.ShapeDtypeStruct((B, Sq, D_MODEL), F32),
        in_specs=[pl.BlockSpec(memory_space=pltpu.VMEM)] * 5,
        out_specs=pl.BlockSpec(memory_space=pltpu.VMEM),
        scratch_shapes=[
            pltpu.VMEM((B, Sq, DQ), BF16),
            pltpu.VMEM((B, Sq, DQE), F32),
            pltpu.VMEM((B, Sq, DQE), BF16),
            pltpu.VMEM((B, R1, BLK, DQE), BF16),
            pltpu.VMEM((B, R2, SEG, DQE), BF16),
            pltpu.VMEM((B, Sq, DQ), BF16),
            pltpu.SemaphoreType.DMA((B, R1)),
            pltpu.SemaphoreType.DMA((B, R1)),
            pltpu.SemaphoreType.DMA((B, R2)),
            pltpu.SemaphoreType.DMA((B, R2)),
            pltpu.SemaphoreType.DMA((B, R2)),
            pltpu.SemaphoreType.DMA((B, R2)),
            pltpu.SemaphoreType.DMA((B, R1)),
            pltpu.SemaphoreType.DMA((B, R1)),
        ],
        compiler_params=pltpu.CompilerParams(collective_id=0),
    )(x, Wq, Wo, K_ext, V_ext)
